# baseline (speedup 1.0000x reference)
import jax
import jax.numpy as jnp
from jax import lax
from jax.experimental import pallas as pl
from jax.experimental.pallas import tpu as pltpu
from jax.experimental.pallas import tpu_sc as plsc

N = 4096
MODES = 16
RES = 64
N0 = 1.5
EY_SIZE = 512
TOTAL = (N + 8) * RES
L = 16
NW = 32
HALO = 8
SEG_F = MODES * EY_SIZE
CH = 6
KQ = EY_SIZE // L
BODY = HALO * RES

SPLIT = 2048
SC_ROWS = (N - SPLIT) // NW
SC_NSEG = SC_ROWS + HALO
SC_ACC_F = BODY + (SC_ROWS + 8) * RES
SC_OUT_F = (N + 8 - SPLIT) * RES - 32


def _sc_body(u_hbm, n_hbm, ey_hbm, out_hbm,
             ustage, nstage, wstage, ey_a, ey_b, acc, sem_a, sem_b):
  cid = lax.axis_index("c")
  sid = lax.axis_index("s")
  wid = cid * 16 + sid
  r0 = SPLIT + wid * SC_ROWS
  s_org = r0 - HALO

  pltpu.sync_copy(u_hbm.at[pl.ds(s_org, SC_NSEG)], ustage)
  pltpu.sync_copy(n_hbm.at[pl.ds(s_org, SC_NSEG)], nstage)

  def wbody(li, _):
    nv = nstage[li]
    wstage[li] = (nv * N0 / (nv + N0)) * ustage[li]
    return 0
  lax.fori_loop(0, SC_NSEG, wbody, 0)

  def zbody(i, _):
    acc[pl.ds(i * L, L)] = jnp.zeros((L,), jnp.float32)
    return 0
  lax.fori_loop(0, SC_ACC_F // L, zbody, 0)

  def ey_copy(seg, buf, sem):
    return pltpu.make_async_copy(ey_hbm.at[pl.ds(seg, CH)], buf, sem)

  def compute(seg, buf, ci):
    li = seg - s_org
    base = li * RES
    wv = wstage[li]

    zero = jnp.zeros((L,), jnp.float32)
    kh = KQ // 2
    for half in range(2):
      koff = half * kh * L

      def mbody(m, accs, koff=koff):
        wm = wv.at[jnp.full((L,), m, jnp.int32)].get(mode="promise_in_bounds")
        return tuple(a + wm * buf[ci, m, pl.ds(koff + k * L, L)]
                     for k, a in enumerate(accs))

      accs = lax.fori_loop(0, MODES, mbody, (zero,) * kh)
      for k in range(kh):
        plsc.addupdate(acc.at[pl.ds(base + koff + k * L, L)], accs[k])

  ey_copy(s_org, ey_a, sem_a).start()
  ey_copy(s_org + CH, ey_b, sem_b).start()

  def chunk(s0c, buf):
    def cbody(c, _):
      compute(s0c + c, buf, c)
      return 0
    lax.fori_loop(0, CH, cbody, 0)

  s_hi = s_org + SC_NSEG

  def pbody(p, _):
    sa = s_org + 2 * p * CH
    ey_copy(sa, ey_a, sem_a).wait()
    chunk(sa, ey_a)

    @pl.when(sa + 2 * CH < s_hi)
    def _():
      ey_copy(sa + 2 * CH, ey_a, sem_a).start()

    sb = sa + CH
    ey_copy(sb, ey_b, sem_b).wait()
    chunk(sb, ey_b)

    @pl.when(sb + 2 * CH < s_hi)
    def _():
      ey_copy(sb + 2 * CH, ey_b, sem_b).start()
    return 0
  lax.fori_loop(0, SC_NSEG // (2 * CH), pbody, 0)

  @pl.when(wid != NW - 1)
  def _():
    pltpu.sync_copy(acc.at[pl.ds(BODY, SC_ROWS * RES)],
                    out_hbm.at[pl.ds(wid * SC_ROWS * RES, SC_ROWS * RES)])

  @pl.when(wid == NW - 1)
  def _():
    pltpu.sync_copy(acc.at[pl.ds(BODY, (SC_ROWS + 8) * RES - 32)],
                    out_hbm.at[pl.ds(wid * SC_ROWS * RES, (SC_ROWS + 8) * RES - 32)])


_sc_call = pl.kernel(
    _sc_body,
    out_type=jax.ShapeDtypeStruct((SC_OUT_F,), jnp.float32),
    mesh=plsc.VectorSubcoreMesh(core_axis_name="c", subcore_axis_name="s"),
    scratch_types=[
        pltpu.VMEM((SC_NSEG, L), jnp.float32),
        pltpu.VMEM((SC_NSEG, L), jnp.float32),
        pltpu.VMEM((SC_NSEG, L), jnp.float32),
        pltpu.VMEM((CH, MODES, EY_SIZE), jnp.float32),
        pltpu.VMEM((CH, MODES, EY_SIZE), jnp.float32),
        pltpu.VMEM((SC_ACC_F,), jnp.float32),
        pltpu.SemaphoreType.DMA,
        pltpu.SemaphoreType.DMA,
    ],
)


TC_B = 64
TC_G = SPLIT // TC_B


def _tc_body(u_ref, n_ref, ey_ref, out_ref, carry):
  g = pl.program_id(0)

  @pl.when(g == 0)
  def _():
    carry[...] = jnp.zeros((8, RES), jnp.float32)

  eta = n_ref[...] * N0 / (n_ref[...] + N0)
  w = eta * u_ref[...]
  ey_sum = jnp.sum(w[:, :, None] * ey_ref[...], axis=1)
  chunks = ey_sum.reshape(TC_B, 8, RES)
  acc = jnp.zeros((TC_B + 7, RES), jnp.float32)
  for c in range(8):
    acc = acc + jnp.pad(chunks[:, c, :], ((c, 7 - c), (0, 0)))
  out_ref[...] = acc[:TC_B] + jnp.pad(carry[:7], ((0, TC_B - 7), (0, 0)))
  carry[:7] = acc[TC_B:]


_tc_call = pl.pallas_call(
    _tc_body,
    grid=(TC_G,),
    in_specs=[
        pl.BlockSpec((TC_B, MODES), lambda g: (g, 0)),
        pl.BlockSpec((TC_B, MODES), lambda g: (g, 0)),
        pl.BlockSpec((TC_B, MODES, EY_SIZE), lambda g: (g, 0, 0)),
    ],
    out_specs=pl.BlockSpec((TC_B, RES), lambda g: (g, 0)),
    out_shape=jax.ShapeDtypeStruct((SPLIT, RES), jnp.float32),
    scratch_shapes=[pltpu.VMEM((8, RES), jnp.float32)],
)


@jax.jit
def kernel(hs, U, neff, Ey):
  del hs
  tc_rows = _tc_call(U, neff, Ey)
  sc_rows = _sc_call(U, neff, Ey)
  zpad = jnp.zeros((RES // 2,), jnp.float32)
  return jnp.concatenate([zpad, tc_rows.reshape(-1), sc_rows])


# speedup vs baseline: 1.1024x; 1.1024x over previous
import jax
import jax.numpy as jnp
from jax import lax
from jax.experimental import pallas as pl
from jax.experimental.pallas import tpu as pltpu
from jax.experimental.pallas import tpu_sc as plsc

N = 4096
MODES = 16
RES = 64
N0 = 1.5
EY_SIZE = 512
TOTAL = (N + 8) * RES
L = 16
NW = 32
HALO = 8
SEG_F = MODES * EY_SIZE
CH = 4
KQ = EY_SIZE // L
BODY = HALO * RES

SPLIT = 2048
SC_ROWS = (N - SPLIT) // NW
SC_NSEG = SC_ROWS + HALO
SC_ACC_F = BODY + (SC_ROWS + 8) * RES
SC_OUT_F = (N + 8 - SPLIT) * RES


def _sc_body(u_hbm, n_hbm, ey_hbm, out_hbm,
             ustage, nstage, wstage, ey_a, ey_b, acc, sem_a, sem_b):
  cid = lax.axis_index("c")
  sid = lax.axis_index("s")
  wid = cid * 16 + sid
  r0 = SPLIT + wid * SC_ROWS
  s_org = r0 - HALO

  pltpu.sync_copy(u_hbm.at[pl.ds(s_org, SC_NSEG)], ustage)
  pltpu.sync_copy(n_hbm.at[pl.ds(s_org, SC_NSEG)], nstage)

  def wbody(li, _):
    nv = nstage[li]
    wstage[li] = (nv * N0 / (nv + N0)) * ustage[li]
    return 0
  lax.fori_loop(0, SC_NSEG, wbody, 0)

  def zbody(i, _):
    acc[pl.ds(i * L, L)] = jnp.zeros((L,), jnp.float32)
    return 0
  lax.fori_loop(0, SC_ACC_F // L, zbody, 0)

  def ey_copy(seg, buf, sem):
    return pltpu.make_async_copy(ey_hbm.at[pl.ds(seg, CH)], buf, sem)

  def compute(seg, buf, ci):
    li = seg - s_org
    base = li * RES
    wv = wstage[li]

    zero = jnp.zeros((L,), jnp.float32)
    kh = KQ // 2
    for half in range(2):
      koff = half * kh * L

      def mbody(m, accs, koff=koff):
        wm = wv.at[jnp.full((L,), m, jnp.int32)].get(mode="promise_in_bounds")
        return tuple(a + wm * buf[ci, m, pl.ds(koff + k * L, L)]
                     for k, a in enumerate(accs))

      accs = lax.fori_loop(0, MODES, mbody, (zero,) * kh)
      for k in range(kh):
        plsc.addupdate(acc.at[pl.ds(base + koff + k * L, L)], accs[k])

  ey_copy(s_org, ey_a, sem_a).start()
  ey_copy(s_org + CH, ey_b, sem_b).start()

  def chunk(s0c, buf):
    def cbody(c, _):
      compute(s0c + c, buf, c)
      return 0
    lax.fori_loop(0, CH, cbody, 0)

  s_hi = s_org + SC_NSEG

  def pbody(p, _):
    sa = s_org + 2 * p * CH
    ey_copy(sa, ey_a, sem_a).wait()
    chunk(sa, ey_a)

    @pl.when(sa + 2 * CH < s_hi)
    def _():
      ey_copy(sa + 2 * CH, ey_a, sem_a).start()

    sb = sa + CH
    ey_copy(sb, ey_b, sem_b).wait()
    chunk(sb, ey_b)

    @pl.when(sb + 2 * CH < s_hi)
    def _():
      ey_copy(sb + 2 * CH, ey_b, sem_b).start()
    return 0
  lax.fori_loop(0, SC_NSEG // (2 * CH), pbody, 0)

  @pl.when(wid != NW - 1)
  def _():
    pltpu.sync_copy(acc.at[pl.ds(BODY, SC_ROWS * RES)],
                    out_hbm.at[pl.ds(wid * SC_ROWS * RES, SC_ROWS * RES)])

  @pl.when(wid == NW - 1)
  def _():
    pltpu.sync_copy(acc.at[pl.ds(BODY, (SC_ROWS + 8) * RES)],
                    out_hbm.at[pl.ds(wid * SC_ROWS * RES, (SC_ROWS + 8) * RES)])


_sc_call = pl.kernel(
    _sc_body,
    out_type=jax.ShapeDtypeStruct((SC_OUT_F,), jnp.float32),
    mesh=plsc.VectorSubcoreMesh(core_axis_name="c", subcore_axis_name="s"),
    scratch_types=[
        pltpu.VMEM((SC_NSEG, L), jnp.float32),
        pltpu.VMEM((SC_NSEG, L), jnp.float32),
        pltpu.VMEM((SC_NSEG, L), jnp.float32),
        pltpu.VMEM((CH, MODES, EY_SIZE), jnp.float32),
        pltpu.VMEM((CH, MODES, EY_SIZE), jnp.float32),
        pltpu.VMEM((SC_ACC_F,), jnp.float32),
        pltpu.SemaphoreType.DMA,
        pltpu.SemaphoreType.DMA,
    ],
)


TC_B = 64
TC_G = SPLIT // TC_B


def _tc_body(u_ref, n_ref, ey_ref, out_ref, carry):
  g = pl.program_id(0)

  @pl.when(g == 0)
  def _():
    carry[...] = jnp.zeros((8, RES), jnp.float32)

  eta = n_ref[...] * N0 / (n_ref[...] + N0)
  w = eta * u_ref[...]
  ey_sum = jnp.sum(w[:, :, None] * ey_ref[...], axis=1)
  chunks = ey_sum.reshape(TC_B, 8, RES)
  acc = jnp.zeros((TC_B + 7, RES), jnp.float32)
  for c in range(8):
    acc = acc + jnp.pad(chunks[:, c, :], ((c, 7 - c), (0, 0)))
  out_ref[...] = acc[:TC_B] + jnp.pad(carry[:7], ((0, TC_B - 7), (0, 0)))
  carry[:7] = acc[TC_B:]


_tc_call = pl.pallas_call(
    _tc_body,
    grid=(TC_G,),
    in_specs=[
        pl.BlockSpec((TC_B, MODES), lambda g: (g, 0)),
        pl.BlockSpec((TC_B, MODES), lambda g: (g, 0)),
        pl.BlockSpec((TC_B, MODES, EY_SIZE), lambda g: (g, 0, 0)),
    ],
    out_specs=pl.BlockSpec((TC_B, RES), lambda g: (g, 0)),
    out_shape=jax.ShapeDtypeStruct((SPLIT, RES), jnp.float32),
    scratch_shapes=[pltpu.VMEM((8, RES), jnp.float32)],
)


@jax.jit
def kernel(hs, U, neff, Ey):
  del hs
  tc_rows = _tc_call(U, neff, Ey)
  sc_rows = _sc_call(U, neff, Ey)
  en_p = jnp.concatenate([tc_rows.reshape(-1), sc_rows])
  return jnp.roll(en_p, RES // 2)
